# baseline (speedup 1.0000x reference)
import functools

import jax
import jax.numpy as jnp
from jax import lax
from jax.experimental import pallas as pl
from jax.experimental.pallas import tpu as pltpu
from jax.experimental.pallas import tpu_sc as plsc


def _make_sc_gather(V, D, B):
    info = plsc.get_sparse_core_info()
    NC, NS = info.num_cores, info.num_subcores
    NW = NC * NS
    assert B % (8 * NW) == 0
    b_per_w = B // NW
    nch = 4
    chunk = b_per_w // nch

    mesh = plsc.VectorSubcoreMesh(core_axis_name="c", subcore_axis_name="s")

    @functools.partial(
        pl.kernel,
        mesh=mesh,
        out_type=jax.ShapeDtypeStruct((NW, nch, chunk, D), jnp.float32),
        scratch_types=[
            pltpu.VMEM((nch, chunk), jnp.int32),
            pltpu.VMEM((nch, chunk, D), jnp.float32),
            pltpu.SemaphoreType.DMA((nch,)),
            pltpu.SemaphoreType.DMA((2,)),
        ],
    )
    def gather_kernel(idx_hbm, table_hbm, out_hbm, idx_v, rows_v, gsem, ssem):
        wid = lax.axis_index("s") * NC + lax.axis_index("c")

        def gather(c):
            return pltpu.async_copy(
                table_hbm.at[idx_v.at[c]], rows_v.at[c], gsem.at[c]
            )

        pltpu.sync_copy(idx_hbm.at[wid, pl.ds(0, 2)], idx_v.at[pl.ds(0, 2)])
        g01 = [gather(0), gather(1)]
        pltpu.sync_copy(idx_hbm.at[wid, pl.ds(2, 2)], idx_v.at[pl.ds(2, 2)])
        g23 = [gather(2), gather(3)]
        for g in g01:
            g.wait()
        s0 = pltpu.async_copy(
            rows_v.at[pl.ds(0, 2)], out_hbm.at[wid, pl.ds(0, 2)], ssem.at[0]
        )
        for g in g23:
            g.wait()
        s1 = pltpu.async_copy(
            rows_v.at[pl.ds(2, 2)], out_hbm.at[wid, pl.ds(2, 2)], ssem.at[1]
        )
        s0.wait()
        s1.wait()

    return gather_kernel


def kernel(x, boxes_weight):
    V, D = boxes_weight.shape
    (B,) = x.shape
    fn = _make_sc_gather(V, D, B)
    info = plsc.get_sparse_core_info()
    nw = info.num_cores * info.num_subcores
    b_per_w = B // nw
    x2 = x.astype(jnp.int32).reshape(nw, 4, b_per_w // 4)
    out = fn(x2, boxes_weight)
    return out.reshape(B, D)


# speedup vs baseline: 1.0062x; 1.0062x over previous
import functools

import jax
import jax.numpy as jnp
from jax import lax
from jax.experimental import pallas as pl
from jax.experimental.pallas import tpu as pltpu
from jax.experimental.pallas import tpu_sc as plsc


def _make_sc_gather(V, D, B):
    info = plsc.get_sparse_core_info()
    NC, NS = info.num_cores, info.num_subcores
    NW = NC * NS
    assert B % (8 * NW) == 0
    b_per_w = B // NW

    mesh = plsc.VectorSubcoreMesh(core_axis_name="c", subcore_axis_name="s")

    @functools.partial(
        pl.kernel,
        mesh=mesh,
        out_type=jax.ShapeDtypeStruct((B, D), jnp.float32),
        scratch_types=[
            pltpu.VMEM((b_per_w,), jnp.int32),
            pltpu.VMEM((b_per_w, D), jnp.float32),
            pltpu.SemaphoreType.DMA,
        ],
    )
    def gather_kernel(idx_hbm, table_hbm, out_hbm, idx_v, rows_v, sem):
        wid = lax.axis_index("s") * NC + lax.axis_index("c")
        base = wid * b_per_w
        pltpu.sync_copy(idx_hbm.at[pl.ds(base, b_per_w)], idx_v)
        pltpu.async_copy(table_hbm.at[idx_v], rows_v, sem).wait()
        pltpu.sync_copy(rows_v, out_hbm.at[pl.ds(base, b_per_w)])

    return gather_kernel


def kernel(x, boxes_weight):
    V, D = boxes_weight.shape
    (B,) = x.shape
    fn = _make_sc_gather(V, D, B)
    return fn(x.astype(jnp.int32), boxes_weight)
